# baseline (speedup 1.0000x reference)
import functools

import jax
import jax.numpy as jnp
from jax.experimental import pallas as pl

_Q = 1024
_D = 512
_BQ = 256


def _retrieve(qb, p):
    sim = jax.lax.dot_general(
        qb, p, (((1,), (1,)), ((), ())), preferred_element_type=jnp.float32)
    m = jnp.max(sim, axis=-1, keepdims=True)
    e = jnp.exp(sim - m)
    s = jnp.sum(e, axis=-1, keepdims=True)
    num = jax.lax.dot_general(
        e.astype(jnp.bfloat16), p, (((1,), (0,)), ((), ())),
        preferred_element_type=jnp.float32)
    return num * (1.0 / s)


def _body(qb_ref, pg_ref, pa_ref, pb_ref, w1_ref, b1_ref, w2t_ref, b2_ref,
          o_ref):
    qb = qb_ref[...]
    rg = _retrieve(qb, pg_ref[...])
    ra = _retrieve(qb, pa_ref[...])
    rb = _retrieve(qb, pb_ref[...])
    cr = 0.5 * (ra + rb)

    comb = jnp.concatenate([cr, rg], axis=-1)
    h = jax.lax.dot_general(
        comb.astype(jnp.bfloat16), w1_ref[...], (((1,), (0,)), ((), ())),
        preferred_element_type=jnp.float32) + b1_ref[...]
    h = 0.5 * h * (1.0 + jax.lax.erf(h * 0.7071067811865476))
    logit = jnp.sum(h * w2t_ref[...], axis=-1, keepdims=True) + b2_ref[...]
    gate = jax.nn.sigmoid(logit)
    o_ref[...] = gate * cr + (1.0 - gate) * rg


@functools.partial(jax.jit, static_argnames=())
def kernel(query, global_patterns, classA_patterns, classB_patterns,
           W1, b1, W2, b2):
    kg = global_patterns.shape[0]
    kc = classA_patterns.shape[0]
    grid = (_Q // _BQ,)
    out = pl.pallas_call(
        _body,
        grid=grid,
        in_specs=[
            pl.BlockSpec((_BQ, _D), lambda i: (i, 0)),
            pl.BlockSpec((kg, _D), lambda i: (0, 0)),
            pl.BlockSpec((kc, _D), lambda i: (0, 0)),
            pl.BlockSpec((kc, _D), lambda i: (0, 0)),
            pl.BlockSpec((2 * _D, 64), lambda i: (0, 0)),
            pl.BlockSpec((1, 64), lambda i: (0, 0)),
            pl.BlockSpec((1, 64), lambda i: (0, 0)),
            pl.BlockSpec((1, 1), lambda i: (0, 0)),
        ],
        out_specs=pl.BlockSpec((_BQ, _D), lambda i: (i, 0)),
        out_shape=jax.ShapeDtypeStruct((_Q, _D), jnp.float32),
    )(query.astype(jnp.bfloat16),
      global_patterns.astype(jnp.bfloat16),
      classA_patterns.astype(jnp.bfloat16),
      classB_patterns.astype(jnp.bfloat16),
      W1.astype(jnp.bfloat16),
      b1.reshape(1, 64), W2.reshape(1, 64), b2.reshape(1, 1))
    return out


# speedup vs baseline: 1.3101x; 1.3101x over previous
import functools

import jax
import jax.numpy as jnp
from jax.experimental import pallas as pl

_Q = 1024
_D = 512
_BQ = 256


def _retrieve(qb, p):
    sim = jax.lax.dot_general(
        qb, p, (((1,), (1,)), ((), ())), preferred_element_type=jnp.float32,
        precision=jax.lax.Precision.DEFAULT)
    m = jnp.max(sim, axis=-1, keepdims=True)
    e = jnp.exp(sim - m)
    s = jnp.sum(e, axis=-1, keepdims=True)
    num = jax.lax.dot_general(
        e, p, (((1,), (0,)), ((), ())), preferred_element_type=jnp.float32,
        precision=jax.lax.Precision.DEFAULT)
    return num * (1.0 / s)


def _body(qb_ref, pg_ref, pa_ref, pb_ref, w1_ref, b1_ref, w2t_ref, b2_ref,
          o_ref):
    qb = qb_ref[...]
    rg = _retrieve(qb, pg_ref[...])
    ra = _retrieve(qb, pa_ref[...])
    rb = _retrieve(qb, pb_ref[...])
    cr = 0.5 * (ra + rb)

    comb = jnp.concatenate([cr, rg], axis=-1)
    h = jax.lax.dot_general(
        comb, w1_ref[...], (((1,), (0,)), ((), ())),
        preferred_element_type=jnp.float32,
        precision=jax.lax.Precision.DEFAULT) + b1_ref[...]
    h = 0.5 * h * (1.0 + jax.lax.erf(h * 0.7071067811865476))
    logit = jnp.sum(h * w2t_ref[...], axis=-1, keepdims=True) + b2_ref[...]
    gate = jax.nn.sigmoid(logit)
    o_ref[...] = gate * cr + (1.0 - gate) * rg


@functools.partial(jax.jit, static_argnames=())
def kernel(query, global_patterns, classA_patterns, classB_patterns,
           W1, b1, W2, b2):
    kg = global_patterns.shape[0]
    kc = classA_patterns.shape[0]
    grid = (_Q // _BQ,)
    out = pl.pallas_call(
        _body,
        grid=grid,
        in_specs=[
            pl.BlockSpec((_BQ, _D), lambda i: (i, 0)),
            pl.BlockSpec((kg, _D), lambda i: (0, 0)),
            pl.BlockSpec((kc, _D), lambda i: (0, 0)),
            pl.BlockSpec((kc, _D), lambda i: (0, 0)),
            pl.BlockSpec((2 * _D, 64), lambda i: (0, 0)),
            pl.BlockSpec((1, 64), lambda i: (0, 0)),
            pl.BlockSpec((1, 64), lambda i: (0, 0)),
            pl.BlockSpec((1, 1), lambda i: (0, 0)),
        ],
        out_specs=pl.BlockSpec((_BQ, _D), lambda i: (i, 0)),
        out_shape=jax.ShapeDtypeStruct((_Q, _D), jnp.float32),
    )(query, global_patterns, classA_patterns, classB_patterns,
      W1, b1.reshape(1, 64), W2.reshape(1, 64), b2.reshape(1, 1))
    return out


# speedup vs baseline: 1.4396x; 1.0988x over previous
import functools

import jax
import jax.numpy as jnp
from jax.experimental import pallas as pl

_Q = 1024
_D = 512
_BQ = 512


def _retrieve(qb, p):
    sim = jax.lax.dot_general(
        qb, p, (((1,), (1,)), ((), ())), preferred_element_type=jnp.float32,
        precision=jax.lax.Precision.DEFAULT)
    m = jnp.max(sim, axis=-1, keepdims=True)
    e = jnp.exp(sim - m)
    s = jnp.sum(e, axis=-1, keepdims=True)
    num = jax.lax.dot_general(
        e, p, (((1,), (0,)), ((), ())), preferred_element_type=jnp.float32,
        precision=jax.lax.Precision.DEFAULT)
    return num * (1.0 / s)


def _body(qb_ref, pg_ref, pa_ref, pb_ref, w1_ref, b1_ref, w2t_ref, b2_ref,
          o_ref):
    qb = qb_ref[...]
    rg = _retrieve(qb, pg_ref[...])
    ra = _retrieve(qb, pa_ref[...])
    rb = _retrieve(qb, pb_ref[...])
    cr = 0.5 * (ra + rb)

    comb = jnp.concatenate([cr, rg], axis=-1)
    h = jax.lax.dot_general(
        comb, w1_ref[...], (((1,), (0,)), ((), ())),
        preferred_element_type=jnp.float32,
        precision=jax.lax.Precision.DEFAULT) + b1_ref[...]
    h = 0.5 * h * (1.0 + jax.lax.erf(h * 0.7071067811865476))
    logit = jnp.sum(h * w2t_ref[...], axis=-1, keepdims=True) + b2_ref[...]
    gate = jax.nn.sigmoid(logit)
    o_ref[...] = gate * cr + (1.0 - gate) * rg


@functools.partial(jax.jit, static_argnames=())
def kernel(query, global_patterns, classA_patterns, classB_patterns,
           W1, b1, W2, b2):
    kg = global_patterns.shape[0]
    kc = classA_patterns.shape[0]
    grid = (_Q // _BQ,)
    out = pl.pallas_call(
        _body,
        grid=grid,
        in_specs=[
            pl.BlockSpec((_BQ, _D), lambda i: (i, 0)),
            pl.BlockSpec((kg, _D), lambda i: (0, 0)),
            pl.BlockSpec((kc, _D), lambda i: (0, 0)),
            pl.BlockSpec((kc, _D), lambda i: (0, 0)),
            pl.BlockSpec((2 * _D, 64), lambda i: (0, 0)),
            pl.BlockSpec((1, 64), lambda i: (0, 0)),
            pl.BlockSpec((1, 64), lambda i: (0, 0)),
            pl.BlockSpec((1, 1), lambda i: (0, 0)),
        ],
        out_specs=pl.BlockSpec((_BQ, _D), lambda i: (i, 0)),
        out_shape=jax.ShapeDtypeStruct((_Q, _D), jnp.float32),
    )(query, global_patterns, classA_patterns, classB_patterns,
      W1, b1.reshape(1, 64), W2.reshape(1, 64), b2.reshape(1, 1))
    return out
